# Initial kernel scaffold; baseline (speedup 1.0000x reference)
import jax
import jax.numpy as jnp
from jax.experimental import pallas as pl


def kernel(x, edge_index, batch, W1, b1, W2, b2, Wfc, bfc):
    raise NotImplementedError("write your pallas kernel here")


# speedup vs baseline: 27.0651x; 27.0651x over previous
import functools

import jax
import jax.numpy as jnp
from jax import lax
from jax.experimental import pallas as pl
from jax.experimental.pallas import tpu as pltpu
from jax.experimental.pallas import tpu_sc as plsc

N = 10000
E = 320000
D = 128
H = 32
G = 64

NCORES = 2
NSUB = 16
NW = NCORES * NSUB
CH = 128
NCH = -(-E // (NW * CH))
EPW = NCH * CH
EPAD = NW * EPW
NPAD = 10240
RPT = NPAD // NSUB
DEGW = 16

def _deg_body(dst_hbm, zeros_hbm, out_hbm, dst_v, ones_v, deg_sh):
    cid = lax.axis_index("c")
    sid = lax.axis_index("s")
    wid = sid * NCORES + cid
    pltpu.sync_copy(dst_hbm.at[wid], dst_v)

    def _fill(r, carry):
        ones_v[r, :] = jnp.ones((DEGW,), jnp.float32)
        return carry

    lax.fori_loop(0, CH, _fill, 0)
    pltpu.sync_copy(zeros_hbm.at[pl.ds(sid * RPT, RPT)],
                    deg_sh.at[pl.ds(sid * RPT, RPT)])
    plsc.subcore_barrier()

    def _scat(j, carry):
        pltpu.sync_copy(ones_v, deg_sh.at[dst_v.at[j]], add=True)
        return carry

    lax.fori_loop(0, NCH, _scat, 0)
    plsc.subcore_barrier()
    pltpu.sync_copy(deg_sh.at[pl.ds(sid * RPT, RPT)],
                    out_hbm.at[pl.ds(cid * NPAD + sid * RPT, RPT)])


def _msg_body(h_hbm, src_hbm, dst_hbm, zeros_hbm, out_hbm,
              src_v, dst_v, rows_v, acc_sh, sem):
    cid = lax.axis_index("c")
    sid = lax.axis_index("s")
    wid = sid * NCORES + cid
    pltpu.sync_copy(src_hbm.at[wid], src_v)
    pltpu.sync_copy(dst_hbm.at[wid], dst_v)
    pltpu.sync_copy(zeros_hbm.at[pl.ds(sid * RPT, RPT)],
                    acc_sh.at[pl.ds(sid * RPT, RPT)])
    plsc.subcore_barrier()

    def _step(j, carry):
        pltpu.async_copy(h_hbm.at[src_v.at[j]], rows_v, sem).wait()
        pltpu.sync_copy(rows_v, acc_sh.at[dst_v.at[j]], add=True)
        return carry

    lax.fori_loop(0, NCH, _step, 0)
    plsc.subcore_barrier()
    pltpu.sync_copy(acc_sh.at[pl.ds(sid * RPT, RPT)],
                    out_hbm.at[pl.ds(cid * NPAD + sid * RPT, RPT)])


@functools.cache
def _build_sc_kernels():
    mesh = plsc.VectorSubcoreMesh(core_axis_name="c", subcore_axis_name="s",
                                  num_cores=NCORES, num_subcores=NSUB)
    params = pltpu.CompilerParams(use_tc_tiling_on_sc=False)
    deg_kernel = pl.kernel(
        _deg_body,
        out_type=jax.ShapeDtypeStruct((NCORES * NPAD, DEGW), jnp.float32),
        mesh=mesh,
        compiler_params=params,
        scratch_types=[
            pltpu.VMEM((NCH, CH), jnp.int32),
            pltpu.VMEM((CH, DEGW), jnp.float32),
            pltpu.VMEM_SHARED((NPAD, DEGW), jnp.float32),
        ],
    )
    msg_kernel = pl.kernel(
        _msg_body,
        out_type=jax.ShapeDtypeStruct((NCORES * NPAD, H), jnp.float32),
        mesh=mesh,
        compiler_params=params,
        scratch_types=[
            pltpu.VMEM((NCH, CH), jnp.int32),
            pltpu.VMEM((NCH, CH), jnp.int32),
            pltpu.VMEM((CH, H), jnp.float32),
            pltpu.VMEM_SHARED((NPAD, H), jnp.float32),
            pltpu.SemaphoreType.DMA,
        ],
    )
    return deg_kernel, msg_kernel


def _prep_body(x_ref, w1_ref, degp_ref, h1p_ref, dinv_ref):
    deg = jnp.sum(degp_ref[0] + degp_ref[1], axis=1, keepdims=True) * (1.0 / DEGW)
    dinv = lax.rsqrt(deg + 1.0)
    t1 = jnp.dot(x_ref[...], w1_ref[...], preferred_element_type=jnp.float32)
    h1p_ref[...] = t1 * dinv
    dinv_ref[...] = dinv


_prep_call = pl.pallas_call(
    _prep_body,
    out_shape=(jax.ShapeDtypeStruct((NPAD, H), jnp.float32),
               jax.ShapeDtypeStruct((NPAD, 1), jnp.float32)),
)


def _mid_body(accp_ref, hp_ref, dinv_ref, w2_ref, b1_ref, out_ref):
    a = (accp_ref[0] + accp_ref[1] + hp_ref[...]) * dinv_ref[...] + b1_ref[...]
    h = jnp.maximum(a, 0.0)
    out_ref[...] = jnp.dot(h, w2_ref[...],
                           preferred_element_type=jnp.float32) * dinv_ref[...]


_mid_call = pl.pallas_call(
    _mid_body,
    out_shape=jax.ShapeDtypeStruct((NPAD, H), jnp.float32),
)


def _fin_body(accp_ref, hp_ref, dinv_ref, b2_ref, wfc_ref, bfc_ref, batch_ref,
              out_ref):
    a = (accp_ref[0] + accp_ref[1] + hp_ref[...]) * dinv_ref[...] + b2_ref[...]
    h = jnp.maximum(a, 0.0)
    p = jnp.dot(h, wfc_ref[...], preferred_element_type=jnp.float32)
    oh = (batch_ref[...] == lax.broadcasted_iota(jnp.int32, (NPAD, G), 1))
    ohf = oh.astype(jnp.float32)
    sums = jnp.sum(ohf * p, axis=0)
    counts = jnp.sum(ohf, axis=0)
    out_ref[...] = sums / jnp.maximum(counts, 1.0) + bfc_ref[...]


_fin_call = pl.pallas_call(
    _fin_body,
    out_shape=jax.ShapeDtypeStruct((G,), jnp.float32),
)


def kernel(x, edge_index, batch, W1, b1, W2, b2, Wfc, bfc):
    src = edge_index[0]
    dst = edge_index[1]
    fill = jnp.full((EPAD - E,), NPAD - 1, dtype=jnp.int32)
    srcp = jnp.concatenate([src, fill]).reshape(NW, NCH, CH)
    dstp = jnp.concatenate([dst, fill]).reshape(NW, NCH, CH)
    xpad = jnp.pad(x, ((0, NPAD - N), (0, 0)))
    batchp = jnp.pad(batch, (0, NPAD - N), constant_values=G).reshape(NPAD, 1)
    zd = jnp.zeros((NPAD, DEGW), jnp.float32)
    zh = jnp.zeros((NPAD, H), jnp.float32)

    deg_kernel, msg_kernel = _build_sc_kernels()
    degp = deg_kernel(dstp, zd).reshape(NCORES, NPAD, DEGW)
    h1p, dinv = _prep_call(xpad, W1, degp)
    acc1 = msg_kernel(h1p, srcp, dstp, zh).reshape(NCORES, NPAD, H)
    h2p = _mid_call(acc1, h1p, dinv, W2, b1)
    acc2 = msg_kernel(h2p, srcp, dstp, zh).reshape(NCORES, NPAD, H)
    return _fin_call(acc2, h2p, dinv, b2, Wfc, bfc, batchp)


# speedup vs baseline: 45.1720x; 1.6690x over previous
import functools

import jax
import jax.numpy as jnp
from jax import lax
from jax.experimental import pallas as pl
from jax.experimental.pallas import tpu as pltpu
from jax.experimental.pallas import tpu_sc as plsc

N = 10000
E = 320000
D = 128
H = 32
G = 64

NCORES = 2
NSUB = 16
NW = NCORES * NSUB
CH = 128
NCH = 80
EPW = NCH * CH
EPAD = NW * EPW
NPAD = 10240
RPT = NPAD // NSUB
DEGW = 16

def _deg_body(dst_hbm, zeros_hbm, out_hbm, dst_v, ones_v, deg_sh):
    cid = lax.axis_index("c")
    sid = lax.axis_index("s")
    wid = sid * NCORES + cid
    pltpu.sync_copy(dst_hbm.at[wid], dst_v)

    def _fill(r, carry):
        ones_v[r, :] = jnp.ones((DEGW,), jnp.float32)
        return carry

    lax.fori_loop(0, CH, _fill, 0)
    pltpu.sync_copy(zeros_hbm.at[pl.ds(sid * RPT, RPT)],
                    deg_sh.at[pl.ds(sid * RPT, RPT)])
    plsc.subcore_barrier()

    def _scat(j, carry):
        pltpu.sync_copy(ones_v, deg_sh.at[dst_v.at[j]], add=True)
        return carry

    lax.fori_loop(0, NCH, _scat, 0)
    plsc.subcore_barrier()
    pltpu.sync_copy(deg_sh.at[pl.ds(sid * RPT, RPT)],
                    out_hbm.at[pl.ds(cid * NPAD + sid * RPT, RPT)])


def _msg_body(h_hbm, src_hbm, dst_hbm, zeros_hbm, out_hbm,
              src_v, dst_v, buf0, buf1, h_sh, acc_sh, sem0, sem1):
    cid = lax.axis_index("c")
    sid = lax.axis_index("s")
    wid = sid * NCORES + cid
    pltpu.sync_copy(src_hbm.at[wid], src_v)
    pltpu.sync_copy(dst_hbm.at[wid], dst_v)
    sl = pl.ds(sid * RPT, RPT)
    pltpu.sync_copy(zeros_hbm.at[sl], acc_sh.at[sl])
    pltpu.sync_copy(h_hbm.at[sl], h_sh.at[sl])
    plsc.subcore_barrier()
    pltpu.async_copy(h_sh.at[src_v.at[0]], buf0, sem0)

    def _pair(k, carry):
        j = 2 * k
        pltpu.make_async_copy(h_sh.at[src_v.at[j]], buf0, sem0).wait()
        pltpu.async_copy(h_sh.at[src_v.at[j + 1]], buf1, sem1)
        pltpu.sync_copy(buf0, acc_sh.at[dst_v.at[j]], add=True)
        pltpu.make_async_copy(h_sh.at[src_v.at[j + 1]], buf1, sem1).wait()

        @pl.when(k < NCH // 2 - 1)
        def _prefetch():
            pltpu.async_copy(h_sh.at[src_v.at[j + 2]], buf0, sem0)

        pltpu.sync_copy(buf1, acc_sh.at[dst_v.at[j + 1]], add=True)
        return carry

    lax.fori_loop(0, NCH // 2, _pair, 0)
    plsc.subcore_barrier()
    pltpu.sync_copy(acc_sh.at[sl],
                    out_hbm.at[pl.ds(cid * NPAD + sid * RPT, RPT)])


@functools.cache
def _build_sc_kernels():
    mesh = plsc.VectorSubcoreMesh(core_axis_name="c", subcore_axis_name="s",
                                  num_cores=NCORES, num_subcores=NSUB)
    params = pltpu.CompilerParams(use_tc_tiling_on_sc=False)
    deg_kernel = pl.kernel(
        _deg_body,
        out_type=jax.ShapeDtypeStruct((NCORES * NPAD, DEGW), jnp.float32),
        mesh=mesh,
        compiler_params=params,
        scratch_types=[
            pltpu.VMEM((NCH, CH), jnp.int32),
            pltpu.VMEM((CH, DEGW), jnp.float32),
            pltpu.VMEM_SHARED((NPAD, DEGW), jnp.float32),
        ],
    )
    msg_kernel = pl.kernel(
        _msg_body,
        out_type=jax.ShapeDtypeStruct((NCORES * NPAD, H), jnp.float32),
        mesh=mesh,
        compiler_params=params,
        scratch_types=[
            pltpu.VMEM((NCH, CH), jnp.int32),
            pltpu.VMEM((NCH, CH), jnp.int32),
            pltpu.VMEM((CH, H), jnp.float32),
            pltpu.VMEM((CH, H), jnp.float32),
            pltpu.VMEM_SHARED((NPAD, H), jnp.float32),
            pltpu.VMEM_SHARED((NPAD, H), jnp.float32),
            pltpu.SemaphoreType.DMA,
            pltpu.SemaphoreType.DMA,
        ],
    )
    return deg_kernel, msg_kernel


def _prep_body(x_ref, w1_ref, degp_ref, h1p_ref, dinv_ref):
    deg = jnp.sum(degp_ref[0] + degp_ref[1], axis=1, keepdims=True) * (1.0 / DEGW)
    dinv = lax.rsqrt(deg + 1.0)
    t1 = jnp.dot(x_ref[...], w1_ref[...], preferred_element_type=jnp.float32)
    h1p_ref[...] = t1 * dinv
    dinv_ref[...] = dinv


_prep_call = pl.pallas_call(
    _prep_body,
    out_shape=(jax.ShapeDtypeStruct((NPAD, H), jnp.float32),
               jax.ShapeDtypeStruct((NPAD, 1), jnp.float32)),
)


def _mid_body(accp_ref, hp_ref, dinv_ref, w2_ref, b1_ref, out_ref):
    a = (accp_ref[0] + accp_ref[1] + hp_ref[...]) * dinv_ref[...] + b1_ref[...]
    h = jnp.maximum(a, 0.0)
    out_ref[...] = jnp.dot(h, w2_ref[...],
                           preferred_element_type=jnp.float32) * dinv_ref[...]


_mid_call = pl.pallas_call(
    _mid_body,
    out_shape=jax.ShapeDtypeStruct((NPAD, H), jnp.float32),
)


def _fin_body(accp_ref, hp_ref, dinv_ref, b2_ref, wfc_ref, bfc_ref, batch_ref,
              out_ref):
    a = (accp_ref[0] + accp_ref[1] + hp_ref[...]) * dinv_ref[...] + b2_ref[...]
    h = jnp.maximum(a, 0.0)
    p = jnp.dot(h, wfc_ref[...], preferred_element_type=jnp.float32)
    oh = (batch_ref[...] == lax.broadcasted_iota(jnp.int32, (NPAD, G), 1))
    ohf = oh.astype(jnp.float32)
    sums = jnp.sum(ohf * p, axis=0)
    counts = jnp.sum(ohf, axis=0)
    out_ref[...] = sums / jnp.maximum(counts, 1.0) + bfc_ref[...]


_fin_call = pl.pallas_call(
    _fin_body,
    out_shape=jax.ShapeDtypeStruct((G,), jnp.float32),
)


def kernel(x, edge_index, batch, W1, b1, W2, b2, Wfc, bfc):
    src = edge_index[0]
    dst = edge_index[1]
    fill = jnp.full((EPAD - E,), NPAD - 1, dtype=jnp.int32)
    srcp = jnp.concatenate([src, fill]).reshape(NW, NCH, CH)
    dstp = jnp.concatenate([dst, fill]).reshape(NW, NCH, CH)
    xpad = jnp.pad(x, ((0, NPAD - N), (0, 0)))
    batchp = jnp.pad(batch, (0, NPAD - N), constant_values=G).reshape(NPAD, 1)
    zd = jnp.zeros((NPAD, DEGW), jnp.float32)
    zh = jnp.zeros((NPAD, H), jnp.float32)

    deg_kernel, msg_kernel = _build_sc_kernels()
    degp = deg_kernel(dstp, zd).reshape(NCORES, NPAD, DEGW)
    h1p, dinv = _prep_call(xpad, W1, degp)
    acc1 = msg_kernel(h1p, srcp, dstp, zh).reshape(NCORES, NPAD, H)
    h2p = _mid_call(acc1, h1p, dinv, W2, b1)
    acc2 = msg_kernel(h2p, srcp, dstp, zh).reshape(NCORES, NPAD, H)
    return _fin_call(acc2, h2p, dinv, b2, Wfc, bfc, batchp)
